# Initial kernel scaffold; baseline (speedup 1.0000x reference)
import jax
import jax.numpy as jnp
from jax.experimental import pallas as pl


def kernel(x, centroid_lut, conv_lut, add_lut, relu_lut, weights):
    raise NotImplementedError("write your pallas kernel here")


# speedup vs baseline: 1.0008x; 1.0008x over previous
import jax
import jax.numpy as jnp
from jax.experimental import pallas as pl

_K = 256
_IMG = 64
_CFG = [
    (3, 64, 7, 0, 4, True),
    (64, 64, 3, 1, 1, True),
    (64, 64, 3, 1, 1, False),
    (64, 64, 3, 1, 1, True),
    (64, 64, 3, 1, 1, False),
    (64, 128, 3, 1, 2, True),
    (128, 128, 3, 1, 1, False),
    (128, 128, 3, 1, 1, True),
    (128, 128, 3, 1, 1, False),
    (128, 256, 3, 1, 2, True),
    (256, 256, 3, 1, 1, False),
    (256, 256, 3, 1, 1, True),
    (256, 256, 3, 1, 1, False),
    (256, 512, 3, 1, 2, True),
    (512, 512, 3, 1, 1, False),
    (512, 512, 3, 1, 1, True),
    (512, 512, 3, 1, 1, False),
]


def _windows(sym, kk, stride, pad):
    if pad > 0:
        sym = jnp.pad(sym, ((pad, pad), (pad, pad), (0, 0)))
    H, W, C = sym.shape
    oh = (H - kk) // stride + 1
    ow = (W - kk) // stride + 1
    ii = jnp.arange(oh)[:, None] * stride + jnp.arange(kk)[None, :]
    jj = jnp.arange(ow)[:, None] * stride + jnp.arange(kk)[None, :]
    win = sym[ii[:, None, :, None], jj[None, :, None, :], :]
    return win.reshape(oh * ow, kk * kk * C), oh, ow


def _mean_kernel(inp_ref, out_ref):
    out_ref[...] = jnp.mean(inp_ref[...], axis=0)


def kernel(x, centroid_lut, conv_lut, add_lut, relu_lut, weights):
    img = jnp.transpose(x, (1, 2, 0))
    sym = jnp.argmin(
        jnp.abs(img[..., None] - centroid_lut[None, None, None, :]), axis=-1
    ).astype(jnp.int32)
    for w, (cin, cout, kk, pad, st, rl) in zip(weights, _CFG):
        patches, oh, ow = _windows(sym, kk, st, pad)
        mult = conv_lut[patches[:, :, None], w[None, :, :]]

        def step(acc, nxt):
            return add_lut[acc, nxt], None

        acc, _ = jax.lax.scan(step, mult[:, 0, :], jnp.moveaxis(mult[:, 1:, :], 1, 0))
        sym = acc.reshape(oh, ow, w.shape[1])
        if rl:
            sym = relu_lut[sym]
    out_img = centroid_lut[sym]
    flat = out_img.reshape(-1, 512)
    return pl.pallas_call(
        _mean_kernel,
        out_shape=jax.ShapeDtypeStruct((512,), jnp.float32),
    )(flat)


# speedup vs baseline: 128.4940x; 128.3970x over previous
import jax
import jax.numpy as jnp
import numpy as np
from jax import lax
from jax.experimental import pallas as pl
from jax.experimental.pallas import tpu as pltpu
from jax.experimental.pallas import tpu_sc as plsc

NC, NS = 2, 16
NW = NC * NS
K = 256
IMG = 64

_CFG = [
    (3, 64, 7, 0, 4, True),
    (64, 64, 3, 1, 1, True),
    (64, 64, 3, 1, 1, False),
    (64, 64, 3, 1, 1, True),
    (64, 64, 3, 1, 1, False),
    (64, 128, 3, 1, 2, True),
    (128, 128, 3, 1, 1, False),
    (128, 128, 3, 1, 1, True),
    (128, 128, 3, 1, 1, False),
    (128, 256, 3, 1, 2, True),
    (256, 256, 3, 1, 1, False),
    (256, 256, 3, 1, 1, True),
    (256, 256, 3, 1, 1, False),
    (256, 512, 3, 1, 2, True),
    (512, 512, 3, 1, 1, False),
    (512, 512, 3, 1, 1, True),
    (512, 512, 3, 1, 1, False),
]


def _plans():
    plans = []
    H = IMG
    for (cin, cout, kk, pad, st, rl) in _CFG:
        Hp = H + 2 * pad
        oh = (Hp - kk) // st + 1
        wins = oh * oh
        S = kk * kk * cin
        RL = kk * cin
        o_cnt = None
        for cand in (128, 64):
            if cout < cand:
                continue
            o_slices = cout // cand
            if o_slices > NW or NW % o_slices:
                continue
            if NW // o_slices <= wins:
                o_cnt = cand
                break
        assert o_cnt is not None
        o_slices = cout // o_cnt
        w_groups = NW // o_slices
        if RL % 16:
            CS = RL
        else:
            CS = 16
            for cand in (128, 112, 96, 80, 64, 48, 32, 16):
                if RL % cand == 0 and cand * o_cnt <= 8192:
                    CS = cand
                    break
        n_chunks = S // CS
        assert n_chunks * CS == S
        q, r = divmod(wins, w_groups)
        n_w_max = q + (1 if r else 0)
        if r == 0 and q % oh == 0:
            wrows_max = q // oh
        else:
            wrows_max = (oh - 1 + n_w_max - 1) // oh + 1
        BH = (wrows_max - 1) * st + kk
        plans.append(dict(
            cin=cin, cout=cout, kk=kk, pad=pad, st=st, rl=rl,
            H=H, Hp=Hp, oh=oh, wins=wins, S=S, RL=RL,
            o_cnt=o_cnt, o_slices=o_slices, w_groups=w_groups,
            CS=CS, n_chunks=n_chunks, CPR=RL // CS, q=q, r=r,
            n_w_max=n_w_max, BH=BH, BAND=BH * Hp * cin,
        ))
        H = oh
    return plans


_PLANS = _plans()

_mesh_cache = []


def _get_mesh():
    if not _mesh_cache:
        _mesh_cache.append(plsc.VectorSubcoreMesh(
            core_axis_name="c", subcore_axis_name="s",
            num_cores=NC, num_subcores=NS))
    return _mesh_cache[0]


def _wid():
    return lax.axis_index("s") * NC + lax.axis_index("c")


def _disc_body(xf, cent, out, cent_v, x_v, s_v):
    wid = _wid()
    n = (IMG * IMG * 3) // NW
    pltpu.sync_copy(cent, cent_v)
    pltpu.sync_copy(xf.at[pl.ds(wid * n, n)], x_v)

    def gf(idx):
        return plsc.bitcast(plsc.load_gather(cent_v, [idx]), jnp.float32)

    for i in range(n // 16):
        xv = x_v[pl.ds(i * 16, 16)]
        pos = jnp.zeros((16,), jnp.int32)
        for b in (128, 64, 32, 16, 8, 4, 2, 1):
            cb = gf(pos + (b - 1))
            pos = pos + jnp.where(cb < xv, b, 0)
        lo = jnp.maximum(pos - 1, 0)
        hi = jnp.minimum(pos, K - 1)
        cl = gf(lo)
        ch = gf(hi)
        s_v[pl.ds(i * 16, 16)] = jnp.where(
            jnp.abs(xv - cl) <= jnp.abs(xv - ch), lo, hi)
    pltpu.sync_copy(s_v, out.at[pl.ds(wid * n, n)])


def _log2(n):
    b = n.bit_length() - 1
    assert (1 << b) == n
    return b


def _make_layer_body(p):
    cin, cout, kk, pad, st, rl = (p["cin"], p["cout"], p["kk"], p["pad"],
                                  p["st"], p["rl"])
    Hp, oh, S, RL = p["Hp"], p["oh"], p["S"], p["RL"]
    o_cnt, w_groups = p["o_cnt"], p["w_groups"]
    CS, n_chunks, CPR, q, r = p["CS"], p["n_chunks"], p["CPR"], p["q"], p["r"]
    G = o_cnt // 16
    CSo = CS * o_cnt
    Hin = Hp - 2 * pad
    rowlen = Hin * cin
    dstride = Hp * cin

    def body(sym, wgt, convp, addp, relu, out,
             conv_v, add_v, relu_v, pad_v, wbuf, out_v):
        wid = _wid()
        if w_groups == NW:
            o_slice = jnp.int32(0)
            gw = wid
        else:
            o_slice = lax.shift_right_logical(wid, _log2(w_groups))
            gw = jnp.bitwise_and(wid, w_groups - 1)
        w_base = gw * q + jnp.minimum(gw, r)
        n_w = q + jnp.where(gw < r, 1, 0)
        o_base = o_slice * o_cnt
        if oh & (oh - 1) == 0:
            i0 = lax.shift_right_logical(w_base, _log2(oh))
        else:
            i0 = (w_base.astype(jnp.float32)
                  * np.float32(1.0 / oh)).astype(jnp.int32)
        j0 = w_base - i0 * oh

        pltpu.sync_copy(convp, conv_v)
        pltpu.sync_copy(addp, add_v)
        if rl:
            pltpu.sync_copy(relu, relu_v)

        BH, BAND = p["BH"], p["BAND"]
        r_lo = i0 * st
        zero16 = jnp.zeros((16,), jnp.int32)

        def zbody(i, c):
            pad_v[pl.ds(i * 16, 16)] = zero16
            return c
        lax.fori_loop(0, BAND // 16, zbody, 0, unroll=4)

        for k in range(BH):
            rr = r_lo + k

            @pl.when(jnp.logical_and(rr >= pad, rr < Hp - pad))
            def _():
                pltpu.sync_copy(
                    sym.at[pl.ds((rr - pad) * rowlen, rowlen)],
                    pad_v.at[pl.ds(k * dstride + pad * cin, rowlen)])

        def chain_steps(pv, nsteps, sl_base, accs):
            P2v = lax.shift_left(lax.shift_right_logical(pv, 2), 8)
            SHv = lax.shift_left(jnp.bitwise_and(pv, 3), 3)
            for j in range(nsteps):
                P2 = P2v[j]
                SH = SHv[j]
                mults = []
                for g in range(G):
                    wv = wbuf[pl.ds((sl_base + j) * o_cnt + g * 16, 16)]
                    cw = plsc.load_gather(conv_v, [P2 + wv])
                    mults.append(jnp.bitwise_and(
                        lax.shift_right_logical(cw, SH), 255))
                if accs is None:
                    accs = tuple(mults)
                else:
                    new = []
                    for a, m in zip(accs, mults):
                        aidx = lax.shift_left(
                            lax.shift_right_logical(a, 1), 8) + m
                        aw = plsc.load_gather(add_v, [aidx])
                        sh = lax.shift_left(jnp.bitwise_and(a, 1), 4)
                        new.append(jnp.bitwise_and(
                            lax.shift_right_logical(aw, sh), 255))
                    accs = tuple(new)
            return accs

        def load_accs(w):
            return tuple(out_v[pl.ds(w * o_cnt + g * 16, 16)]
                         for g in range(G))

        def store_accs(w, accs):
            for g in range(G):
                out_v[pl.ds(w * o_cnt + g * 16, 16)] = accs[g]

        def run_chunk(c, di, cir, first):
            pltpu.sync_copy(
                wgt.at[pl.ds((o_slice * n_chunks + c) * CSo, CSo)], wbuf)
            soff = di * dstride + cir * CS

            def wbody(w, carry):
                i, j = carry
                base_w = ((i - i0) * (st * Hp) + j * st) * cin
                rs = base_w + soff
                if CS % 16:
                    idx = lax.iota(jnp.int32, 16)
                    pv0 = plsc.load_gather(pad_v, [idx + rs])
                    pv1 = plsc.load_gather(pad_v, [idx + (rs + 16)])
                    accs = chain_steps(pv0, 16, 0, None if first else
                                       load_accs(w))
                    accs = chain_steps(pv1, CS - 16, 16, accs)
                else:
                    if first:
                        pv = pad_v[pl.ds(rs, 16)]
                        accs = chain_steps(pv, 16, 0, None)
                        b0 = 1
                    else:
                        accs = load_accs(w)
                        b0 = 0

                    def bbody(b, accs):
                        pv = pad_v[pl.ds(rs + b * 16, 16)]
                        return chain_steps(pv, 16, b * 16, accs)
                    accs = lax.fori_loop(b0, CS // 16, bbody, accs)
                store_accs(w, accs)
                j1 = j + 1
                wrap = j1 == oh
                return (jnp.where(wrap, i + 1, i), jnp.where(wrap, 0, j1))
            lax.fori_loop(0, n_w, wbody, (i0, j0))

        run_chunk(0, jnp.int32(0), jnp.int32(0), True)

        def cbody(c, carry):
            di, cir = carry
            cir1 = cir + 1
            wrap = cir1 == CPR
            di = jnp.where(wrap, di + 1, di)
            cir = jnp.where(wrap, 0, cir1)
            run_chunk(c, di, cir, False)
            return (di, cir)
        if n_chunks > 1:
            lax.fori_loop(1, n_chunks, cbody, (jnp.int32(0), jnp.int32(0)))

        def obody(w, carry):
            if rl:
                for g in range(G):
                    a = out_v[pl.ds(w * o_cnt + g * 16, 16)]
                    out_v[pl.ds(w * o_cnt + g * 16, 16)] = \
                        plsc.load_gather(relu_v, [a])
            pltpu.sync_copy(
                out_v.at[pl.ds(w * o_cnt, o_cnt)],
                out.at[pl.ds((w_base + w) * cout + o_base, o_cnt)])
            return carry
        lax.fori_loop(0, n_w, obody, 0)

    return body


def _mean_body(symf, cent, out, cent_v, s_v, o_v):
    wid = _wid()
    pltpu.sync_copy(cent, cent_v)
    for w in range(4):
        pltpu.sync_copy(symf.at[pl.ds(w * 512 + wid * 16, 16)],
                        s_v.at[pl.ds(w * 16, 16)])
    acc = jnp.zeros((16,), jnp.float32)
    for w in range(4):
        sv = s_v[pl.ds(w * 16, 16)]
        acc = acc + plsc.bitcast(plsc.load_gather(cent_v, [sv]), jnp.float32)
    o_v[...] = acc * 0.25
    pltpu.sync_copy(o_v, out.at[pl.ds(wid * 16, 16)])


def _vmem(n, dt=jnp.int32):
    return pltpu.VMEM((n,), dt)


_CPARAMS = pltpu.CompilerParams(needs_layout_passes=False)


def kernel(x, centroid_lut, conv_lut, add_lut, relu_lut, weights):
    xf = jnp.transpose(x, (1, 2, 0)).reshape(-1)
    cent_bits = lax.bitcast_convert_type(centroid_lut, jnp.int32)
    c4 = conv_lut.reshape(64, 4, K)
    convp = (c4[:, 0] | (c4[:, 1] << 8) | (c4[:, 2] << 16)
             | (c4[:, 3] << 24)).reshape(-1)
    a2 = add_lut.reshape(128, 2, K)
    addp = (a2[:, 0] | (a2[:, 1] << 16)).reshape(-1)

    wgts = []
    for p, wl in zip(_PLANS, weights):
        wgts.append(wl.reshape(p["n_chunks"], p["CS"],
                               p["o_slices"], p["o_cnt"])
                    .transpose(2, 0, 1, 3).reshape(-1))

    disc = pl.kernel(
        _disc_body,
        out_type=jax.ShapeDtypeStruct((IMG * IMG * 3,), jnp.int32),
        mesh=_get_mesh(),
        compiler_params=_CPARAMS,
        scratch_types=[_vmem(K), _vmem(384, jnp.float32),
                       _vmem(384)],
    )
    sym = disc(xf, cent_bits)

    for i, p in enumerate(_PLANS):
        layer = pl.kernel(
            _make_layer_body(p),
            out_type=jax.ShapeDtypeStruct((p["wins"] * p["cout"],), jnp.int32),
            mesh=_get_mesh(),
            compiler_params=_CPARAMS,
            scratch_types=[
                _vmem(16384),
                _vmem(32768),
                _vmem(K),
                _vmem(p["BAND"]),
                _vmem(p["CS"] * p["o_cnt"]),
                _vmem(p["n_w_max"] * p["o_cnt"]),
            ],
        )
        sym = layer(sym, wgts[i], convp, addp, relu_lut)

    mean = pl.kernel(
        _mean_body,
        out_type=jax.ShapeDtypeStruct((512,), jnp.float32),
        mesh=_get_mesh(),
        compiler_params=_CPARAMS,
        scratch_types=[_vmem(K), _vmem(64), _vmem(16, jnp.float32)],
    )
    return mean(sym, cent_bits)


# speedup vs baseline: 130.8333x; 1.0182x over previous
import jax
import jax.numpy as jnp
import numpy as np
from jax import lax
from jax.experimental import pallas as pl
from jax.experimental.pallas import tpu as pltpu
from jax.experimental.pallas import tpu_sc as plsc

NC, NS = 2, 16
NW = NC * NS
K = 256
IMG = 64

_CFG = [
    (3, 64, 7, 0, 4, True),
    (64, 64, 3, 1, 1, True),
    (64, 64, 3, 1, 1, False),
    (64, 64, 3, 1, 1, True),
    (64, 64, 3, 1, 1, False),
    (64, 128, 3, 1, 2, True),
    (128, 128, 3, 1, 1, False),
    (128, 128, 3, 1, 1, True),
    (128, 128, 3, 1, 1, False),
    (128, 256, 3, 1, 2, True),
    (256, 256, 3, 1, 1, False),
    (256, 256, 3, 1, 1, True),
    (256, 256, 3, 1, 1, False),
    (256, 512, 3, 1, 2, True),
    (512, 512, 3, 1, 1, False),
    (512, 512, 3, 1, 1, True),
    (512, 512, 3, 1, 1, False),
]


def _plans():
    plans = []
    H = IMG
    for (cin, cout, kk, pad, st, rl) in _CFG:
        Hp = H + 2 * pad
        oh = (Hp - kk) // st + 1
        wins = oh * oh
        S = kk * kk * cin
        RL = kk * cin
        o_cnt = None
        for cand in (128, 64):
            if cout < cand:
                continue
            o_slices = cout // cand
            if o_slices > NW or NW % o_slices:
                continue
            if NW // o_slices <= wins:
                o_cnt = cand
                break
        assert o_cnt is not None
        o_slices = cout // o_cnt
        w_groups = NW // o_slices
        if RL % 16:
            CS = RL
        else:
            CS = 16
            for cand in (128, 112, 96, 80, 64, 48, 32, 16):
                if RL % cand == 0 and cand * o_cnt <= 8192:
                    CS = cand
                    break
        n_chunks = S // CS
        assert n_chunks * CS == S
        q, r = divmod(wins, w_groups)
        n_w_max = q + (1 if r else 0)
        magic = -(-(1 << 16) // oh)
        assert all((w * magic) >> 16 == w // oh for w in range(wins))
        if r == 0 and q % oh == 0:
            wrows_max = q // oh
        else:
            wrows_max = (oh - 1 + n_w_max - 1) // oh + 1
        BH = (wrows_max - 1) * st + kk
        plans.append(dict(
            cin=cin, cout=cout, kk=kk, pad=pad, st=st, rl=rl,
            H=H, Hp=Hp, oh=oh, wins=wins, S=S, RL=RL,
            o_cnt=o_cnt, o_slices=o_slices, w_groups=w_groups,
            CS=CS, n_chunks=n_chunks, CPR=RL // CS, q=q, r=r,
            n_w_max=n_w_max, BH=BH, BAND=BH * Hp * cin, magic=magic,
        ))
        H = oh
    return plans


_PLANS = _plans()

_mesh_cache = []


def _get_mesh():
    if not _mesh_cache:
        _mesh_cache.append(plsc.VectorSubcoreMesh(
            core_axis_name="c", subcore_axis_name="s",
            num_cores=NC, num_subcores=NS))
    return _mesh_cache[0]


def _wid():
    return lax.axis_index("s") * NC + lax.axis_index("c")


def _disc_body(xf, cent, out, cent_v, x_v, s_v):
    wid = _wid()
    n = (IMG * IMG * 3) // NW
    pltpu.sync_copy(cent, cent_v)
    pltpu.sync_copy(xf.at[pl.ds(wid * n, n)], x_v)

    def gf(idx):
        return plsc.bitcast(plsc.load_gather(cent_v, [idx]), jnp.float32)

    for i in range(n // 16):
        xv = x_v[pl.ds(i * 16, 16)]
        pos = jnp.zeros((16,), jnp.int32)
        for b in (128, 64, 32, 16, 8, 4, 2, 1):
            cb = gf(pos + (b - 1))
            pos = pos + jnp.where(cb < xv, b, 0)
        lo = jnp.maximum(pos - 1, 0)
        hi = jnp.minimum(pos, K - 1)
        cl = gf(lo)
        ch = gf(hi)
        s_v[pl.ds(i * 16, 16)] = jnp.where(
            jnp.abs(xv - cl) <= jnp.abs(xv - ch), lo, hi)
    pltpu.sync_copy(s_v, out.at[pl.ds(wid * n, n)])


def _log2(n):
    b = n.bit_length() - 1
    assert (1 << b) == n
    return b


def _make_layer_body(p):
    cin, cout, kk, pad, st, rl = (p["cin"], p["cout"], p["kk"], p["pad"],
                                  p["st"], p["rl"])
    Hp, oh, S, RL = p["Hp"], p["oh"], p["S"], p["RL"]
    o_cnt, w_groups = p["o_cnt"], p["w_groups"]
    CS, n_chunks, CPR, q, r = p["CS"], p["n_chunks"], p["CPR"], p["q"], p["r"]
    G = o_cnt // 16
    CSo = CS * o_cnt
    Hin = Hp - 2 * pad
    rowlen = Hin * cin
    dstride = Hp * cin

    def body(sym, wgt, convp, addp, relu, out,
             conv_v, add_v, relu_v, pad_v, wbuf, out_v):
        wid = _wid()
        if w_groups == NW:
            o_slice = jnp.int32(0)
            gw = wid
        else:
            o_slice = lax.shift_right_logical(wid, _log2(w_groups))
            gw = jnp.bitwise_and(wid, w_groups - 1)
        w_base = gw * q + jnp.minimum(gw, r)
        n_w = q + jnp.where(gw < r, 1, 0)
        o_base = o_slice * o_cnt
        if oh & (oh - 1) == 0:
            i0 = lax.shift_right_logical(w_base, _log2(oh))
        else:
            i0 = lax.shift_right_logical(w_base * p["magic"], 16)
        j0 = w_base - i0 * oh

        pltpu.sync_copy(convp, conv_v)
        pltpu.sync_copy(addp, add_v)
        if rl:
            pltpu.sync_copy(relu, relu_v)

        BH, BAND = p["BH"], p["BAND"]
        r_lo = i0 * st
        zero16 = jnp.zeros((16,), jnp.int32)

        def zbody(i, c):
            pad_v[pl.ds(i * 16, 16)] = zero16
            return c
        lax.fori_loop(0, BAND // 16, zbody, 0, unroll=4)

        for k in range(BH):
            rr = r_lo + k

            @pl.when(jnp.logical_and(rr >= pad, rr < Hp - pad))
            def _():
                pltpu.sync_copy(
                    sym.at[pl.ds((rr - pad) * rowlen, rowlen)],
                    pad_v.at[pl.ds(k * dstride + pad * cin, rowlen)])

        def chain_steps(pv, nsteps, sl_base, accs):
            P2v = lax.shift_left(lax.shift_right_logical(pv, 2), 8)
            SHv = lax.shift_left(jnp.bitwise_and(pv, 3), 3)
            for j in range(nsteps):
                P2 = P2v[j]
                SH = SHv[j]
                mults = []
                for g in range(G):
                    wv = wbuf[pl.ds((sl_base + j) * o_cnt + g * 16, 16)]
                    cw = plsc.load_gather(conv_v, [P2 + wv])
                    mults.append(jnp.bitwise_and(
                        lax.shift_right_logical(cw, SH), 255))
                if accs is None:
                    accs = tuple(mults)
                else:
                    accs = tuple(
                        plsc.load_gather(add_v, [lax.shift_left(a, 8) + m])
                        for a, m in zip(accs, mults))
            return accs

        def load_accs(w):
            return tuple(out_v[pl.ds(w * o_cnt + g * 16, 16)]
                         for g in range(G))

        def store_accs(w, accs):
            for g in range(G):
                out_v[pl.ds(w * o_cnt + g * 16, 16)] = accs[g]

        def run_chunk(c, di, cir, first):
            pltpu.sync_copy(
                wgt.at[pl.ds((o_slice * n_chunks + c) * CSo, CSo)], wbuf)
            soff = di * dstride + cir * CS

            def wbody(w, carry):
                i, j = carry
                base_w = ((i - i0) * (st * Hp) + j * st) * cin
                rs = base_w + soff
                if CS % 16:
                    idx = lax.iota(jnp.int32, 16)
                    pv0 = plsc.load_gather(pad_v, [idx + rs])
                    pv1 = plsc.load_gather(pad_v, [idx + (rs + 16)])
                    accs = chain_steps(pv0, 16, 0, None if first else
                                       load_accs(w))
                    accs = chain_steps(pv1, CS - 16, 16, accs)
                else:
                    if first:
                        pv = pad_v[pl.ds(rs, 16)]
                        accs = chain_steps(pv, 16, 0, None)
                        b0 = 1
                    else:
                        accs = load_accs(w)
                        b0 = 0

                    def bbody(b, accs):
                        pv = pad_v[pl.ds(rs + b * 16, 16)]
                        return chain_steps(pv, 16, b * 16, accs)
                    accs = lax.fori_loop(b0, CS // 16, bbody, accs)
                store_accs(w, accs)
                j1 = j + 1
                wrap = j1 == oh
                return (jnp.where(wrap, i + 1, i), jnp.where(wrap, 0, j1))
            lax.fori_loop(0, n_w, wbody, (i0, j0))

        run_chunk(0, jnp.int32(0), jnp.int32(0), True)

        def cbody(c, carry):
            di, cir = carry
            cir1 = cir + 1
            wrap = cir1 == CPR
            di = jnp.where(wrap, di + 1, di)
            cir = jnp.where(wrap, 0, cir1)
            run_chunk(c, di, cir, False)
            return (di, cir)
        if n_chunks > 1:
            lax.fori_loop(1, n_chunks, cbody, (jnp.int32(0), jnp.int32(0)))

        def obody(w, carry):
            if rl:
                for g in range(G):
                    a = out_v[pl.ds(w * o_cnt + g * 16, 16)]
                    out_v[pl.ds(w * o_cnt + g * 16, 16)] = \
                        plsc.load_gather(relu_v, [a])
            pltpu.sync_copy(
                out_v.at[pl.ds(w * o_cnt, o_cnt)],
                out.at[pl.ds((w_base + w) * cout + o_base, o_cnt)])
            return carry
        lax.fori_loop(0, n_w, obody, 0)

    return body


def _mean_body(symf, cent, out, cent_v, s_v, o_v):
    wid = _wid()
    pltpu.sync_copy(cent, cent_v)
    for w in range(4):
        pltpu.sync_copy(symf.at[pl.ds(w * 512 + wid * 16, 16)],
                        s_v.at[pl.ds(w * 16, 16)])
    acc = jnp.zeros((16,), jnp.float32)
    for w in range(4):
        sv = s_v[pl.ds(w * 16, 16)]
        acc = acc + plsc.bitcast(plsc.load_gather(cent_v, [sv]), jnp.float32)
    o_v[...] = acc * 0.25
    pltpu.sync_copy(o_v, out.at[pl.ds(wid * 16, 16)])


def _vmem(n, dt=jnp.int32):
    return pltpu.VMEM((n,), dt)


_CPARAMS = pltpu.CompilerParams(needs_layout_passes=False)


def kernel(x, centroid_lut, conv_lut, add_lut, relu_lut, weights):
    xf = jnp.transpose(x, (1, 2, 0)).reshape(-1)
    cent_bits = lax.bitcast_convert_type(centroid_lut, jnp.int32)
    c4 = conv_lut.reshape(64, 4, K)
    convp = (c4[:, 0] | (c4[:, 1] << 8) | (c4[:, 2] << 16)
             | (c4[:, 3] << 24)).reshape(-1)
    addp = add_lut.reshape(-1)

    wgts = []
    for p, wl in zip(_PLANS, weights):
        wgts.append(wl.reshape(p["n_chunks"], p["CS"],
                               p["o_slices"], p["o_cnt"])
                    .transpose(2, 0, 1, 3).reshape(-1))

    disc = pl.kernel(
        _disc_body,
        out_type=jax.ShapeDtypeStruct((IMG * IMG * 3,), jnp.int32),
        mesh=_get_mesh(),
        compiler_params=_CPARAMS,
        scratch_types=[_vmem(K), _vmem(384, jnp.float32),
                       _vmem(384)],
    )
    sym = disc(xf, cent_bits)

    for i, p in enumerate(_PLANS):
        layer = pl.kernel(
            _make_layer_body(p),
            out_type=jax.ShapeDtypeStruct((p["wins"] * p["cout"],), jnp.int32),
            mesh=_get_mesh(),
            compiler_params=_CPARAMS,
            scratch_types=[
                _vmem(16384),
                _vmem(K * K),
                _vmem(K),
                _vmem(p["BAND"]),
                _vmem(p["CS"] * p["o_cnt"]),
                _vmem(p["n_w_max"] * p["o_cnt"]),
            ],
        )
        sym = layer(sym, wgts[i], convp, addp, relu_lut)

    mean = pl.kernel(
        _mean_body,
        out_type=jax.ShapeDtypeStruct((512,), jnp.float32),
        mesh=_get_mesh(),
        compiler_params=_CPARAMS,
        scratch_types=[_vmem(K), _vmem(64), _vmem(16, jnp.float32)],
    )
    return mean(sym, cent_bits)
